# baseline (speedup 1.0000x reference)
import functools

import jax
import jax.numpy as jnp
from jax import lax
from jax.experimental import pallas as pl
from jax.experimental.pallas import tpu as pltpu
from jax.experimental.pallas import tpu_sc as plsc

NC = 2
NS = 16
NW = NC * NS
L = 16

HIST = 50
CLEN = 4
CP = 8
D_USER = 64
D_ITEM = 64
D_CATE = 32
D = 160
H = 320
NUM_CATES = 1000


def _sc_features(users, items, ihm, ihl, uhm, uhl, cates_p,
                 user_table, item_table, cate_table_flat):
    B = users.shape[0]
    RW = B // NW
    CH = 16
    NCHUNK = RW // CH

    mesh = plsc.VectorSubcoreMesh(core_axis_name="c", subcore_axis_name="s",
                                  num_cores=NC, num_subcores=NS)

    @functools.partial(
        pl.kernel,
        out_type=(jax.ShapeDtypeStruct((B, D), jnp.float32),
                  jax.ShapeDtypeStruct((B, D), jnp.float32)),
        mesh=mesh,
        compiler_params=pltpu.CompilerParams(use_tc_tiling_on_sc=False,
                                             needs_layout_passes=False),
        scratch_types=[
            pltpu.VMEM((NUM_CATES * D_CATE,), jnp.float32),
            pltpu.VMEM((CH, HIST, D_ITEM), jnp.float32),
            pltpu.VMEM((CH, HIST, CP), jnp.int32),
            pltpu.VMEM((CH, HIST), jnp.int32),
            pltpu.VMEM((CH, HIST), jnp.int32),
            pltpu.VMEM((CH,), jnp.int32),
            pltpu.VMEM((CH,), jnp.int32),
            pltpu.VMEM((CH,), jnp.int32),
            pltpu.VMEM((CH,), jnp.int32),
            pltpu.VMEM((CH, D_USER), jnp.float32),
            pltpu.VMEM((CH, D_ITEM), jnp.float32),
            pltpu.VMEM((CH, CP), jnp.int32),
            pltpu.VMEM((CH, D), jnp.float32),
            pltpu.VMEM((CH, D), jnp.float32),
            pltpu.SemaphoreType.DMA,
        ],
    )
    def feat_kernel(users_h, items_h, ihm_h, ihl_h, uhm_h, uhl_h, cp_h,
                    ut_h, it_h, ct_h, uf_h, if_h,
                    ct_v, rows_v, crows_v, ihm_v, uhm_v, ihl_v,
                    uhl_v, users_v, items_v, uemb_v, iemb_v, spk_v,
                    uf_v, if_v, sem):
        wid = lax.axis_index("s") * NC + lax.axis_index("c")
        pltpu.sync_copy(ct_h, ct_v)
        lane = lax.iota(jnp.int32, L)

        def cvec(val):
            return jnp.full((L,), val, jnp.int32)

        def chunk_body(ci, carry):
            base = wid * RW + ci * CH
            pltpu.sync_copy(ihm_h.at[pl.ds(base, CH)], ihm_v)
            pltpu.sync_copy(uhm_h.at[pl.ds(base, CH)], uhm_v)
            pltpu.sync_copy(ihl_h.at[pl.ds(base, CH)], ihl_v)
            pltpu.sync_copy(uhl_h.at[pl.ds(base, CH)], uhl_v)
            pltpu.sync_copy(users_h.at[pl.ds(base, CH)], users_v)
            pltpu.sync_copy(items_h.at[pl.ds(base, CH)], items_v)
            for g in range(2):
                descs = []
                for i in range(g * 8, g * 8 + 8):
                    descs.append(pltpu.async_copy(it_h.at[ihm_v.at[i]], rows_v.at[i], sem))
                    descs.append(pltpu.async_copy(cp_h.at[ihm_v.at[i]], crows_v.at[i], sem))
                for dsc in descs:
                    dsc.wait()
            d_self = [
                pltpu.async_copy(ut_h.at[users_v], uemb_v, sem),
                pltpu.async_copy(it_h.at[items_v], iemb_v, sem),
                pltpu.async_copy(cp_h.at[items_v], spk_v, sem),
            ]

            ihl_i = ihl_v[...]
            inv_ihl = 1.0 / ihl_i.astype(jnp.float32)

            for dg in range(4):
                def abody(t, accs, _dg=dg):
                    w = jnp.where(t < ihl_i, inv_ihl, 0.0)
                    tt = cvec(0) + t
                    out = []
                    for kk in range(16):
                        v = plsc.load_gather(rows_v, [lane, tt, cvec(_dg * 16 + kk)])
                        out.append(accs[kk] + w * v)
                    return tuple(out)
                accs = lax.fori_loop(0, HIST, abody,
                                     (jnp.zeros((L,), jnp.float32),) * 16)
                for kk in range(16):
                    plsc.store_scatter(uf_v, [lane, cvec(64 + dg * 16 + kk)], accs[kk])

            for dg in range(2):
                def cbody(t, accs, _dg=dg):
                    w = jnp.where(t < ihl_i, inv_ihl, 0.0)
                    tt = cvec(0) + t
                    cl = plsc.load_gather(crows_v, [lane, tt, cvec(CLEN)])
                    wc = w / cl.astype(jnp.float32)
                    ids = [plsc.load_gather(crows_v, [lane, tt, cvec(c)]) * D_CATE
                           for c in range(CLEN)]
                    wcs = [jnp.where(c < cl, wc, 0.0) for c in range(CLEN)]
                    out = []
                    for kk in range(16):
                        s = accs[kk]
                        for c in range(CLEN):
                            v = plsc.load_gather(ct_v, [ids[c] + (_dg * 16 + kk)])
                            s = s + wcs[c] * v
                        out.append(s)
                    return tuple(out)
                accs = lax.fori_loop(0, HIST, cbody,
                                     (jnp.zeros((L,), jnp.float32),) * 16)
                for kk in range(16):
                    plsc.store_scatter(uf_v, [lane, cvec(128 + dg * 16 + kk)], accs[kk])

            for dsc in d_self:
                dsc.wait()
            for d in range(D_USER):
                plsc.store_scatter(uf_v, [lane, cvec(d)],
                                   plsc.load_gather(uemb_v, [lane, cvec(d)]))
            for d in range(D_ITEM):
                plsc.store_scatter(if_v, [lane, cvec(d)],
                                   plsc.load_gather(iemb_v, [lane, cvec(d)]))
            scl = plsc.load_gather(spk_v, [lane, cvec(CLEN)])
            sinv = 1.0 / scl.astype(jnp.float32)
            sids = [plsc.load_gather(spk_v, [lane, cvec(c)]) * D_CATE
                    for c in range(CLEN)]
            swcs = [jnp.where(c < scl, sinv, 0.0) for c in range(CLEN)]
            for d in range(D_CATE):
                s = jnp.zeros((L,), jnp.float32)
                for c in range(CLEN):
                    s = s + swcs[c] * plsc.load_gather(ct_v, [sids[c] + d])
                plsc.store_scatter(if_v, [lane, cvec(64 + d)], s)

            d_user = [pltpu.async_copy(ut_h.at[uhm_v.at[i]], rows_v.at[i], sem)
                      for i in range(CH)]
            for dsc in d_user:
                dsc.wait()
            uhl_i = uhl_v[...]
            inv_uhl = 1.0 / uhl_i.astype(jnp.float32)
            for dg in range(4):
                def dbody(t, accs, _dg=dg):
                    w = jnp.where(t < uhl_i, inv_uhl, 0.0)
                    tt = cvec(0) + t
                    out = []
                    for kk in range(16):
                        v = plsc.load_gather(rows_v, [lane, tt, cvec(_dg * 16 + kk)])
                        out.append(accs[kk] + w * v)
                    return tuple(out)
                accs = lax.fori_loop(0, HIST, dbody,
                                     (jnp.zeros((L,), jnp.float32),) * 16)
                for kk in range(16):
                    plsc.store_scatter(if_v, [lane, cvec(96 + dg * 16 + kk)], accs[kk])

            pltpu.sync_copy(uf_v, uf_h.at[pl.ds(base, CH)])
            pltpu.sync_copy(if_v, if_h.at[pl.ds(base, CH)])
            return carry

        lax.fori_loop(0, NCHUNK, chunk_body, 0)

    return feat_kernel(users, items, ihm, ihl, uhm, uhl, cates_p,
                       user_table, item_table, cate_table_flat)


def _prenorm_block(x, g, b, w1, b1, w2, b2):
    m = jnp.mean(x, axis=-1, keepdims=True)
    v = jnp.mean((x - m) ** 2, axis=-1, keepdims=True)
    h = (x - m) / jnp.sqrt(v + 1e-5) * g + b
    h = jnp.maximum(
        jnp.dot(h, w1, preferred_element_type=jnp.float32,
                precision=lax.Precision.HIGHEST) + b1, 0.0)
    h = jnp.dot(h, w2, preferred_element_type=jnp.float32,
                precision=lax.Precision.HIGHEST) + b2
    return h + x


def _l2norm(x):
    n = jnp.sqrt(jnp.sum(x * x, axis=-1, keepdims=True))
    return x / jnp.maximum(n, 1e-12)


def _tc_mlp(uf, itf, params):
    B = uf.shape[0]
    BT = 1024
    grid = B // BT

    names = ("user_mlp", "user_mlp_2", "item_mlp", "item_mlp_2")
    flat_params = []
    for nm in names:
        p = params[nm]
        flat_params += [p["g"].reshape(1, D), p["b"].reshape(1, D),
                        p["w1"], p["b1"].reshape(1, H),
                        p["w2"], p["b2"].reshape(1, D)]

    def body(uf_ref, if_ref, *refs):
        prefs = refs[:24]
        ue_ref, ie_ref = refs[24], refs[25]
        xu = uf_ref[...]
        xi = if_ref[...]

        def block(x, j):
            g, b, w1, b1, w2, b2 = (prefs[6 * j + k][...] for k in range(6))
            return _prenorm_block(x, g, b, w1, b1, w2, b2)

        ue_ref[...] = _l2norm(block(xu, 0) + block(xu, 1))
        ie_ref[...] = _l2norm(block(xi, 2) + block(xi, 3))

    xspec = pl.BlockSpec((BT, D), lambda i: (i, 0))
    pspecs = []
    for p in flat_params:
        pspecs.append(pl.BlockSpec(p.shape, lambda i: (0, 0)))

    return pl.pallas_call(
        body,
        grid=(grid,),
        in_specs=[xspec, xspec] + pspecs,
        out_specs=[xspec, xspec],
        out_shape=(jax.ShapeDtypeStruct((B, D), jnp.float32),
                   jax.ShapeDtypeStruct((B, D), jnp.float32)),
    )(uf, itf, *flat_params)


def kernel(users, items, item_history_matrix, item_history_len,
           user_history_matrix, user_history_len, cates, cate_lens,
           user_table, item_table, cate_table, params):
    cates_p = jnp.concatenate(
        [cates.astype(jnp.int32),
         cate_lens.astype(jnp.int32)[:, None],
         jnp.zeros((cates.shape[0], CP - CLEN - 1), jnp.int32)], axis=1)
    uf, itf = _sc_features(
        users.astype(jnp.int32), items.astype(jnp.int32),
        item_history_matrix.astype(jnp.int32),
        item_history_len.astype(jnp.int32),
        user_history_matrix.astype(jnp.int32),
        user_history_len.astype(jnp.int32),
        cates_p, user_table, item_table, cate_table.reshape(-1))
    return _tc_mlp(uf, itf, params)


# speedup vs baseline: 2.2408x; 2.2408x over previous
import functools

import jax
import jax.numpy as jnp
from jax import lax
from jax.experimental import pallas as pl
from jax.experimental.pallas import tpu as pltpu
from jax.experimental.pallas import tpu_sc as plsc

NC = 2
NS = 16
NW = NC * NS
L = 16

HIST = 50
CLEN = 4
CP = 8
D_USER = 64
D_ITEM = 64
D_CATE = 32
D = 160
H = 320
NUM_CATES = 1000


def _sc_features(users, items, ihm, ihl, uhm, uhl, cates_p,
                 user_table, item_table, cate_table_flat):
    B = users.shape[0]
    RW = B // NW
    CH = 8
    NCHUNK = RW // CH

    mesh = plsc.VectorSubcoreMesh(core_axis_name="c", subcore_axis_name="s",
                                  num_cores=NC, num_subcores=NS)

    @functools.partial(
        pl.kernel,
        out_type=(jax.ShapeDtypeStruct((B, D), jnp.float32),
                  jax.ShapeDtypeStruct((B, D), jnp.float32)),
        mesh=mesh,
        compiler_params=pltpu.CompilerParams(use_tc_tiling_on_sc=False,
                                             needs_layout_passes=False),
        scratch_types=[
            pltpu.VMEM((NUM_CATES * D_CATE,), jnp.float32),
            pltpu.VMEM((CH, HIST, D_ITEM), jnp.float32),
            pltpu.VMEM((CH, HIST, D_USER), jnp.float32),
            pltpu.VMEM((CH + 1, HIST, CP), jnp.int32),
            pltpu.VMEM((CH, HIST), jnp.int32),
            pltpu.VMEM((CH, HIST), jnp.int32),
            pltpu.VMEM((CH + 8,), jnp.int32),
            pltpu.VMEM((CH + 8,), jnp.int32),
            pltpu.VMEM((CH,), jnp.int32),
            pltpu.VMEM((CH,), jnp.int32),
            pltpu.VMEM((CH, D_USER), jnp.float32),
            pltpu.VMEM((CH, D_ITEM), jnp.float32),
            pltpu.VMEM((CH + 1, CP), jnp.int32),
            pltpu.VMEM((CH, D), jnp.float32),
            pltpu.VMEM((CH, D), jnp.float32),
            pltpu.SemaphoreType.DMA,
        ],
    )
    def feat_kernel(users_h, items_h, ihm_h, ihl_h, uhm_h, uhl_h, cp_h,
                    ut_h, it_h, ct_h, uf_h, if_h,
                    ct_v, rows_v, urows_v, crows_v, ihm_v, uhm_v, ihl_v,
                    uhl_v, users_v, items_v, uemb_v, iemb_v, spk_v,
                    uf_v, if_v, sem):
        wid = lax.axis_index("s") * NC + lax.axis_index("c")
        pltpu.sync_copy(ct_h, ct_v)
        lane = lax.iota(jnp.int32, L)

        def cvec(val):
            return jnp.full((L,), val, jnp.int32)

        def splat(ref, idx):
            return plsc.load_gather(ref, [cvec(i) for i in idx])

        def chunk_body(ci, carry):
            base = wid * RW + ci * CH
            pltpu.sync_copy(ihm_h.at[pl.ds(base, CH)], ihm_v)
            pltpu.sync_copy(uhm_h.at[pl.ds(base, CH)], uhm_v)
            pltpu.sync_copy(ihl_h.at[pl.ds(base, CH)], ihl_v.at[pl.ds(8, CH)])
            pltpu.sync_copy(uhl_h.at[pl.ds(base, CH)], uhl_v.at[pl.ds(8, CH)])
            pltpu.sync_copy(users_h.at[pl.ds(base, CH)], users_v)
            pltpu.sync_copy(items_h.at[pl.ds(base, CH)], items_v)
            d_item = []
            for i in range(CH):
                d_item.append(pltpu.async_copy(it_h.at[ihm_v.at[i]], rows_v.at[i], sem))
                d_item.append(pltpu.async_copy(cp_h.at[ihm_v.at[i]], crows_v.at[i + 1], sem))
            for dsc in d_item:
                dsc.wait()
            d_self = [
                pltpu.async_copy(ut_h.at[users_v], uemb_v, sem),
                pltpu.async_copy(it_h.at[items_v], iemb_v, sem),
                pltpu.async_copy(cp_h.at[items_v], spk_v.at[pl.ds(1, CH)], sem),
            ]

            for b in range(CH):
                ihl_b = splat(ihl_v, (8 + b,))
                inv_b = 1.0 / ihl_b.astype(jnp.float32)

                def abody(t, accs, _b=b, _ihl=ihl_b, _inv=inv_b):
                    a0, a1, a2, a3, c0, c1 = accs
                    tv = cvec(0) + t
                    w = jnp.where(tv < _ihl, _inv, 0.0)
                    bv = cvec(_b)
                    a0 = a0 + w * plsc.load_gather(rows_v, [bv, tv, lane])
                    a1 = a1 + w * plsc.load_gather(rows_v, [bv, tv, lane + 16])
                    a2 = a2 + w * plsc.load_gather(rows_v, [bv, tv, lane + 32])
                    a3 = a3 + w * plsc.load_gather(rows_v, [bv, tv, lane + 48])
                    cl = plsc.load_gather(crows_v, [cvec(_b + 1), tv, cvec(CLEN)])
                    wc = w / cl.astype(jnp.float32)
                    for c in range(CLEN):
                        idv = plsc.load_gather(crows_v, [cvec(_b + 1), tv, cvec(c)])
                        wcs = jnp.where(c < cl, wc, 0.0)
                        addr = idv * D_CATE + lane
                        c0 = c0 + wcs * plsc.load_gather(ct_v, [addr])
                        c1 = c1 + wcs * plsc.load_gather(ct_v, [addr + 16])
                    return (a0, a1, a2, a3, c0, c1)

                accs = lax.fori_loop(0, HIST, abody,
                                     (jnp.zeros((L,), jnp.float32),) * 6)
                for dg in range(4):
                    plsc.store_scatter(uf_v, [cvec(b), lane + (64 + dg * 16)], accs[dg])
                plsc.store_scatter(uf_v, [cvec(b), lane + 128], accs[4])
                plsc.store_scatter(uf_v, [cvec(b), lane + 144], accs[5])

            for dsc in d_self:
                dsc.wait()
            for b in range(CH):
                for dg in range(4):
                    plsc.store_scatter(
                        uf_v, [cvec(b), lane + dg * 16],
                        plsc.load_gather(uemb_v, [cvec(b), lane + dg * 16]))
                    plsc.store_scatter(
                        if_v, [cvec(b), lane + dg * 16],
                        plsc.load_gather(iemb_v, [cvec(b), lane + dg * 16]))
                scl = splat(spk_v, (b + 1, CLEN))
                sinv = 1.0 / scl.astype(jnp.float32)
                s0 = jnp.zeros((L,), jnp.float32)
                s1 = jnp.zeros((L,), jnp.float32)
                for c in range(CLEN):
                    idv = splat(spk_v, (b + 1, c))
                    wcs = jnp.where(c < scl, sinv, 0.0)
                    addr = idv * D_CATE + lane
                    s0 = s0 + wcs * plsc.load_gather(ct_v, [addr])
                    s1 = s1 + wcs * plsc.load_gather(ct_v, [addr + 16])
                plsc.store_scatter(if_v, [cvec(b), lane + 64], s0)
                plsc.store_scatter(if_v, [cvec(b), lane + 80], s1)

            d_user = [pltpu.async_copy(ut_h.at[uhm_v.at[i]], urows_v.at[i], sem)
                      for i in range(CH)]
            for dsc in d_user:
                dsc.wait()
            for b in range(CH):
                uhl_b = splat(uhl_v, (8 + b,))
                inv_b = 1.0 / uhl_b.astype(jnp.float32)

                def dbody(t, accs, _b=b, _uhl=uhl_b, _inv=inv_b):
                    a0, a1, a2, a3 = accs
                    tv = cvec(0) + t
                    w = jnp.where(tv < _uhl, _inv, 0.0)
                    bv = cvec(_b)
                    a0 = a0 + w * plsc.load_gather(urows_v, [bv, tv, lane])
                    a1 = a1 + w * plsc.load_gather(urows_v, [bv, tv, lane + 16])
                    a2 = a2 + w * plsc.load_gather(urows_v, [bv, tv, lane + 32])
                    a3 = a3 + w * plsc.load_gather(urows_v, [bv, tv, lane + 48])
                    return (a0, a1, a2, a3)

                accs = lax.fori_loop(0, HIST, dbody,
                                     (jnp.zeros((L,), jnp.float32),) * 4)
                for dg in range(4):
                    plsc.store_scatter(if_v, [cvec(b), lane + (96 + dg * 16)], accs[dg])

            pltpu.sync_copy(uf_v, uf_h.at[pl.ds(base, CH)])
            pltpu.sync_copy(if_v, if_h.at[pl.ds(base, CH)])
            return carry

        lax.fori_loop(0, NCHUNK, chunk_body, 0)

    return feat_kernel(users, items, ihm, ihl, uhm, uhl, cates_p,
                       user_table, item_table, cate_table_flat)


def _prenorm_block(x, g, b, w1, b1, w2, b2):
    m = jnp.mean(x, axis=-1, keepdims=True)
    v = jnp.mean((x - m) ** 2, axis=-1, keepdims=True)
    h = (x - m) / jnp.sqrt(v + 1e-5) * g + b
    h = jnp.maximum(
        jnp.dot(h, w1, preferred_element_type=jnp.float32,
                precision=lax.Precision.HIGHEST) + b1, 0.0)
    h = jnp.dot(h, w2, preferred_element_type=jnp.float32,
                precision=lax.Precision.HIGHEST) + b2
    return h + x


def _l2norm(x):
    n = jnp.sqrt(jnp.sum(x * x, axis=-1, keepdims=True))
    return x / jnp.maximum(n, 1e-12)


def _tc_mlp(uf, itf, params):
    B = uf.shape[0]
    BT = 1024
    grid = B // BT

    names = ("user_mlp", "user_mlp_2", "item_mlp", "item_mlp_2")
    flat_params = []
    for nm in names:
        p = params[nm]
        flat_params += [p["g"].reshape(1, D), p["b"].reshape(1, D),
                        p["w1"], p["b1"].reshape(1, H),
                        p["w2"], p["b2"].reshape(1, D)]

    def body(uf_ref, if_ref, *refs):
        prefs = refs[:24]
        ue_ref, ie_ref = refs[24], refs[25]
        xu = uf_ref[...]
        xi = if_ref[...]

        def block(x, j):
            g, b, w1, b1, w2, b2 = (prefs[6 * j + k][...] for k in range(6))
            return _prenorm_block(x, g, b, w1, b1, w2, b2)

        ue_ref[...] = _l2norm(block(xu, 0) + block(xu, 1))
        ie_ref[...] = _l2norm(block(xi, 2) + block(xi, 3))

    xspec = pl.BlockSpec((BT, D), lambda i: (i, 0))
    pspecs = []
    for p in flat_params:
        pspecs.append(pl.BlockSpec(p.shape, lambda i: (0, 0)))

    return pl.pallas_call(
        body,
        grid=(grid,),
        in_specs=[xspec, xspec] + pspecs,
        out_specs=[xspec, xspec],
        out_shape=(jax.ShapeDtypeStruct((B, D), jnp.float32),
                   jax.ShapeDtypeStruct((B, D), jnp.float32)),
    )(uf, itf, *flat_params)


def kernel(users, items, item_history_matrix, item_history_len,
           user_history_matrix, user_history_len, cates, cate_lens,
           user_table, item_table, cate_table, params):
    cates_p = jnp.concatenate(
        [cates.astype(jnp.int32),
         cate_lens.astype(jnp.int32)[:, None],
         jnp.zeros((cates.shape[0], CP - CLEN - 1), jnp.int32)], axis=1)
    uf, itf = _sc_features(
        users.astype(jnp.int32), items.astype(jnp.int32),
        item_history_matrix.astype(jnp.int32),
        item_history_len.astype(jnp.int32),
        user_history_matrix.astype(jnp.int32),
        user_history_len.astype(jnp.int32),
        cates_p, user_table, item_table, cate_table.reshape(-1))
    return _tc_mlp(uf, itf, params)


# speedup vs baseline: 2.3525x; 1.0499x over previous
import functools

import jax
import jax.numpy as jnp
from jax import lax
from jax.experimental import pallas as pl
from jax.experimental.pallas import tpu as pltpu
from jax.experimental.pallas import tpu_sc as plsc

NC = 2
NS = 16
NW = NC * NS
L = 16

HIST = 50
CLEN = 4
CP = 8
D_USER = 64
D_ITEM = 64
D_CATE = 32
D = 160
H = 320
NUM_CATES = 1000


def _sc_features(users, items, ihm, ihl, uhm, uhl, cates_p,
                 user_table, item_table, cate_table_flat):
    B = users.shape[0]
    RW = B // NW
    CH = 8
    NCHUNK = RW // CH

    mesh = plsc.VectorSubcoreMesh(core_axis_name="c", subcore_axis_name="s",
                                  num_cores=NC, num_subcores=NS)

    @functools.partial(
        pl.kernel,
        out_type=(jax.ShapeDtypeStruct((B, D), jnp.float32),
                  jax.ShapeDtypeStruct((B, D), jnp.float32)),
        mesh=mesh,
        compiler_params=pltpu.CompilerParams(use_tc_tiling_on_sc=False,
                                             needs_layout_passes=False),
        scratch_types=[
            pltpu.VMEM((NUM_CATES * D_CATE,), jnp.float32),
            pltpu.VMEM((CH, HIST, D_ITEM), jnp.float32),
            pltpu.VMEM((CH, HIST, D_USER), jnp.float32),
            pltpu.VMEM((CH + 1, HIST, CP), jnp.int32),
            pltpu.VMEM((CH, HIST), jnp.int32),
            pltpu.VMEM((CH, HIST), jnp.int32),
            pltpu.VMEM((CH + 8,), jnp.int32),
            pltpu.VMEM((CH + 8,), jnp.int32),
            pltpu.VMEM((CH,), jnp.int32),
            pltpu.VMEM((CH,), jnp.int32),
            pltpu.VMEM((CH, D_USER), jnp.float32),
            pltpu.VMEM((CH, D_ITEM), jnp.float32),
            pltpu.VMEM((CH + 1, CP), jnp.int32),
            pltpu.VMEM((CH, D), jnp.float32),
            pltpu.VMEM((CH, D), jnp.float32),
            pltpu.SemaphoreType.DMA,
            pltpu.SemaphoreType.DMA,
            pltpu.SemaphoreType.DMA,
        ],
    )
    def feat_kernel(users_h, items_h, ihm_h, ihl_h, uhm_h, uhl_h, cp_h,
                    ut_h, it_h, ct_h, uf_h, if_h,
                    ct_v, rows_v, urows_v, crows_v, ihm_v, uhm_v, ihl_v,
                    uhl_v, users_v, items_v, uemb_v, iemb_v, spk_v,
                    uf_v, if_v, sem, sem_u, sem_s):
        wid = lax.axis_index("s") * NC + lax.axis_index("c")
        pltpu.sync_copy(ct_h, ct_v)
        lane = lax.iota(jnp.int32, L)

        def cvec(val):
            return jnp.full((L,), val, jnp.int32)

        def splat(ref, idx):
            return plsc.load_gather(ref, [cvec(i) for i in idx])

        def chunk_body(ci, carry):
            base = wid * RW + ci * CH
            pltpu.sync_copy(ihm_h.at[pl.ds(base, CH)], ihm_v)
            pltpu.sync_copy(uhm_h.at[pl.ds(base, CH)], uhm_v)
            pltpu.sync_copy(ihl_h.at[pl.ds(base, CH)], ihl_v.at[pl.ds(8, CH)])
            pltpu.sync_copy(uhl_h.at[pl.ds(base, CH)], uhl_v.at[pl.ds(8, CH)])
            pltpu.sync_copy(users_h.at[pl.ds(base, CH)], users_v)
            pltpu.sync_copy(items_h.at[pl.ds(base, CH)], items_v)
            d_item = []
            for i in range(CH):
                d_item.append(pltpu.async_copy(it_h.at[ihm_v.at[i]], rows_v.at[i], sem))
                d_item.append(pltpu.async_copy(cp_h.at[ihm_v.at[i]], crows_v.at[i + 1], sem))
            for dsc in d_item:
                dsc.wait()
            d_user = [pltpu.async_copy(ut_h.at[uhm_v.at[i]], urows_v.at[i], sem_u)
                      for i in range(CH)]
            d_self = [
                pltpu.async_copy(ut_h.at[users_v], uemb_v, sem_s),
                pltpu.async_copy(it_h.at[items_v], iemb_v, sem_s),
                pltpu.async_copy(cp_h.at[items_v], spk_v.at[pl.ds(1, CH)], sem_s),
            ]

            for b in range(CH):
                ihl_b = splat(ihl_v, (8 + b,))
                inv_b = 1.0 / ihl_b.astype(jnp.float32)

                def abody(t, accs, _b=b, _ihl=ihl_b, _inv=inv_b):
                    a0, a1, a2, a3, c0, c1 = accs
                    tv = cvec(0) + t
                    w = jnp.where(tv < _ihl, _inv, 0.0)
                    bv = cvec(_b)
                    a0 = a0 + w * plsc.load_gather(rows_v, [bv, tv, lane])
                    a1 = a1 + w * plsc.load_gather(rows_v, [bv, tv, lane + 16])
                    a2 = a2 + w * plsc.load_gather(rows_v, [bv, tv, lane + 32])
                    a3 = a3 + w * plsc.load_gather(rows_v, [bv, tv, lane + 48])
                    cl = plsc.load_gather(crows_v, [cvec(_b + 1), tv, cvec(CLEN)])
                    wc = w / cl.astype(jnp.float32)
                    for c in range(CLEN):
                        idv = plsc.load_gather(crows_v, [cvec(_b + 1), tv, cvec(c)])
                        wcs = jnp.where(c < cl, wc, 0.0)
                        addr = idv * D_CATE + lane
                        c0 = c0 + wcs * plsc.load_gather(ct_v, [addr])
                        c1 = c1 + wcs * plsc.load_gather(ct_v, [addr + 16])
                    return (a0, a1, a2, a3, c0, c1)

                accs = lax.fori_loop(0, HIST, abody,
                                     (jnp.zeros((L,), jnp.float32),) * 6)
                for dg in range(4):
                    plsc.store_scatter(uf_v, [cvec(b), lane + (64 + dg * 16)], accs[dg])
                plsc.store_scatter(uf_v, [cvec(b), lane + 128], accs[4])
                plsc.store_scatter(uf_v, [cvec(b), lane + 144], accs[5])

            for dsc in d_self:
                dsc.wait()
            for b in range(CH):
                for dg in range(4):
                    plsc.store_scatter(
                        uf_v, [cvec(b), lane + dg * 16],
                        plsc.load_gather(uemb_v, [cvec(b), lane + dg * 16]))
                    plsc.store_scatter(
                        if_v, [cvec(b), lane + dg * 16],
                        plsc.load_gather(iemb_v, [cvec(b), lane + dg * 16]))
                scl = splat(spk_v, (b + 1, CLEN))
                sinv = 1.0 / scl.astype(jnp.float32)
                s0 = jnp.zeros((L,), jnp.float32)
                s1 = jnp.zeros((L,), jnp.float32)
                for c in range(CLEN):
                    idv = splat(spk_v, (b + 1, c))
                    wcs = jnp.where(c < scl, sinv, 0.0)
                    addr = idv * D_CATE + lane
                    s0 = s0 + wcs * plsc.load_gather(ct_v, [addr])
                    s1 = s1 + wcs * plsc.load_gather(ct_v, [addr + 16])
                plsc.store_scatter(if_v, [cvec(b), lane + 64], s0)
                plsc.store_scatter(if_v, [cvec(b), lane + 80], s1)

            for dsc in d_user:
                dsc.wait()
            for b in range(CH):
                uhl_b = splat(uhl_v, (8 + b,))
                inv_b = 1.0 / uhl_b.astype(jnp.float32)

                def dbody(t, accs, _b=b, _uhl=uhl_b, _inv=inv_b):
                    a0, a1, a2, a3 = accs
                    tv = cvec(0) + t
                    w = jnp.where(tv < _uhl, _inv, 0.0)
                    bv = cvec(_b)
                    a0 = a0 + w * plsc.load_gather(urows_v, [bv, tv, lane])
                    a1 = a1 + w * plsc.load_gather(urows_v, [bv, tv, lane + 16])
                    a2 = a2 + w * plsc.load_gather(urows_v, [bv, tv, lane + 32])
                    a3 = a3 + w * plsc.load_gather(urows_v, [bv, tv, lane + 48])
                    return (a0, a1, a2, a3)

                accs = lax.fori_loop(0, HIST, dbody,
                                     (jnp.zeros((L,), jnp.float32),) * 4)
                for dg in range(4):
                    plsc.store_scatter(if_v, [cvec(b), lane + (96 + dg * 16)], accs[dg])

            pltpu.sync_copy(uf_v, uf_h.at[pl.ds(base, CH)])
            pltpu.sync_copy(if_v, if_h.at[pl.ds(base, CH)])
            return carry

        lax.fori_loop(0, NCHUNK, chunk_body, 0)

    return feat_kernel(users, items, ihm, ihl, uhm, uhl, cates_p,
                       user_table, item_table, cate_table_flat)


def _prenorm_block(x, g, b, w1, b1, w2, b2):
    m = jnp.mean(x, axis=-1, keepdims=True)
    v = jnp.mean((x - m) ** 2, axis=-1, keepdims=True)
    h = (x - m) / jnp.sqrt(v + 1e-5) * g + b
    h = jnp.maximum(
        jnp.dot(h, w1, preferred_element_type=jnp.float32,
                precision=lax.Precision.HIGHEST) + b1, 0.0)
    h = jnp.dot(h, w2, preferred_element_type=jnp.float32,
                precision=lax.Precision.HIGHEST) + b2
    return h + x


def _l2norm(x):
    n = jnp.sqrt(jnp.sum(x * x, axis=-1, keepdims=True))
    return x / jnp.maximum(n, 1e-12)


def _tc_mlp(uf, itf, params):
    B = uf.shape[0]
    BT = 1024
    grid = B // BT

    names = ("user_mlp", "user_mlp_2", "item_mlp", "item_mlp_2")
    flat_params = []
    for nm in names:
        p = params[nm]
        flat_params += [p["g"].reshape(1, D), p["b"].reshape(1, D),
                        p["w1"], p["b1"].reshape(1, H),
                        p["w2"], p["b2"].reshape(1, D)]

    def body(uf_ref, if_ref, *refs):
        prefs = refs[:24]
        ue_ref, ie_ref = refs[24], refs[25]
        xu = uf_ref[...]
        xi = if_ref[...]

        def block(x, j):
            g, b, w1, b1, w2, b2 = (prefs[6 * j + k][...] for k in range(6))
            return _prenorm_block(x, g, b, w1, b1, w2, b2)

        ue_ref[...] = _l2norm(block(xu, 0) + block(xu, 1))
        ie_ref[...] = _l2norm(block(xi, 2) + block(xi, 3))

    xspec = pl.BlockSpec((BT, D), lambda i: (i, 0))
    pspecs = []
    for p in flat_params:
        pspecs.append(pl.BlockSpec(p.shape, lambda i: (0, 0)))

    return pl.pallas_call(
        body,
        grid=(grid,),
        in_specs=[xspec, xspec] + pspecs,
        out_specs=[xspec, xspec],
        out_shape=(jax.ShapeDtypeStruct((B, D), jnp.float32),
                   jax.ShapeDtypeStruct((B, D), jnp.float32)),
    )(uf, itf, *flat_params)


def kernel(users, items, item_history_matrix, item_history_len,
           user_history_matrix, user_history_len, cates, cate_lens,
           user_table, item_table, cate_table, params):
    cates_p = jnp.concatenate(
        [cates.astype(jnp.int32),
         cate_lens.astype(jnp.int32)[:, None],
         jnp.zeros((cates.shape[0], CP - CLEN - 1), jnp.int32)], axis=1)
    uf, itf = _sc_features(
        users.astype(jnp.int32), items.astype(jnp.int32),
        item_history_matrix.astype(jnp.int32),
        item_history_len.astype(jnp.int32),
        user_history_matrix.astype(jnp.int32),
        user_history_len.astype(jnp.int32),
        cates_p, user_table, item_table, cate_table.reshape(-1))
    return _tc_mlp(uf, itf, params)
